# baseline (speedup 1.0000x reference)
import functools

import jax
import jax.numpy as jnp
import numpy as np
from jax import lax
from jax.experimental import pallas as pl
from jax.experimental.pallas import tpu as pltpu
from jax.experimental.pallas import tpu_sc as plsc

N_NODES = 10000
N_EDGES = 320000
D = 128

_Z = np.int32(0)

_NC = 2
_NS = 16
_L = 16
_NW = _NC * _NS
_EPW = N_EDGES // _NW
_C = 80
_NCHUNK = _EPW // _C
_NPAD = 10240
_SLAB = _NPAD // _NS


def _sc_body(q2_hbm, ef_hbm, dst_hbm, mix_out, den_out,
             dst0, dst1, dsts0, dsts1, efw0, efw1, q2r0, q2r1, t0, t1,
             spmix, spden,
             semd0, semd1, semg0, semg1, seme0, seme1,
             semt0, semt1, semm0, semm1):
    c = lax.axis_index("c")
    s = lax.axis_index("s")
    wid = s * np.int32(_NC) + c
    row0 = s * np.int32(_SLAB)
    dst_b = (dst0, dst1)
    dsts_b = (dsts0, dsts1)
    efw_b = (efw0, efw1)
    q2r_b = (q2r0, q2r1)
    t_b = (t0, t1)
    semd_b = (semd0, semd1)
    semg_b = (semg0, semg1)
    seme_b = (seme0, seme1)
    semt_b = (semt0, semt1)
    semm_b = (semm0, semm1)

    def _zrow(j, carry):
        for k in range(8):
            efw0[j, pl.ds(k * _L, _L)] = jnp.zeros((_L,), jnp.float32)
        return carry

    lax.fori_loop(np.int32(0), np.int32(_C), _zrow, np.int32(0))
    for g in range(_C // _L):
        t0[pl.ds(g * _L, _L)] = jnp.zeros((_L,), jnp.float32)
    for j in range(_SLAB // _C):
        pltpu.sync_copy(efw0, spmix.at[pl.ds(row0 + np.int32(j * _C), _C), :])
        pltpu.sync_copy(t0, spden.at[pl.ds(row0 + np.int32(j * _C), _C)])
    plsc.subcore_barrier()

    base_w = wid * np.int32(_EPW)
    lane = lax.iota(jnp.int32, _L)
    perms = [(lane + np.int32(sh)) & np.int32(_L - 1) for sh in (8, 4, 2, 1)]
    dnums = lax.GatherDimensionNumbers(
        offset_dims=(), collapsed_slice_dims=(0,), start_index_map=(0,))

    def _hsum(v):
        for pm in perms:
            v = v + lax.gather(v, pm[:, None], dnums, (1,),
                               mode=lax.GatherScatterMode.PROMISE_IN_BOUNDS)
        return v

    def _ebase(k):
        return base_w + k * np.int32(_C)

    def _issue_loads(k, b):
        pltpu.async_copy(q2_hbm.at[dst_b[b]], q2r_b[b], semg_b[b])
        pltpu.async_copy(ef_hbm.at[pl.ds(_ebase(k), _C), :], efw_b[b], seme_b[b])

    def _wait_loads(b):
        pltpu.make_async_copy(q2_hbm.at[dst_b[b]], q2r_b[b], semg_b[b]).wait()
        pltpu.make_async_copy(ef_hbm.at[pl.ds(np.int32(0), _C), :], efw_b[b],
                              seme_b[b]).wait()

    def _issue_dst(k, b):
        pltpu.async_copy(dst_hbm.at[pl.ds(_ebase(k), _C)], dst_b[b], semd_b[b])

    def _wait_dst(b):
        pltpu.make_async_copy(dst_hbm.at[pl.ds(np.int32(0), _C)], dst_b[b],
                              semd_b[b]).wait()

    def _wait_scatters(b):
        pltpu.make_async_copy(t_b[b], spden.at[dsts_b[b]], semt_b[b]).wait()
        pltpu.make_async_copy(efw_b[b], spmix.at[dsts_b[b]], semm_b[b]).wait()

    def _compute(b):
        efw_v = efw_b[b]
        q2r_v = q2r_b[b]
        t_v = t_b[b]

        def _grp(g, cc):
            e0 = g * np.int32(_L)

            def _edge(e, tvec):
                row = e0 + e
                r = [efw_v[row, pl.ds(k * _L, _L)] for k in range(D // _L)]
                acc = None
                for k in range(D // (2 * _L)):
                    qw = q2r_v[row, pl.ds(k * _L, _L)]
                    qa = lax.bitcast_convert_type(
                        lax.shift_left(qw, np.int32(16)), jnp.float32)
                    qb = lax.bitcast_convert_type(
                        qw & np.int32(-65536), jnp.float32)
                    term = qa * r[2 * k] + qb * r[2 * k + 1]
                    acc = term if acc is None else acc + term
                exv = jnp.exp(jnp.minimum(_hsum(acc), np.float32(80.0)))
                for k in range(D // _L):
                    efw_v[row, pl.ds(k * _L, _L)] = r[k] * exv
                return jnp.where(lane == e, exv, tvec)

            tvec = lax.fori_loop(np.int32(0), np.int32(_L), _edge,
                                 jnp.zeros((_L,), jnp.float32))
            t_v[pl.ds(e0, _L)] = tvec
            return cc

        lax.fori_loop(np.int32(0), np.int32(_C // _L), _grp, np.int32(0))

    def _issue_scatters(b):
        for g in range(_C // _L):
            sl = pl.ds(g * _L, _L)
            dsts_b[b][sl] = dst_b[b][sl]
        pltpu.async_copy(t_b[b], spden.at[dsts_b[b]], semt_b[b], add=True)
        pltpu.async_copy(efw_b[b], spmix.at[dsts_b[b]], semm_b[b], add=True)

    def _body(k, b, wait_scat):
        nb = 1 - b
        if wait_scat:
            _wait_scatters(nb)
        _wait_loads(b)
        _wait_dst(nb)
        _issue_loads(k + np.int32(1), nb)
        _compute(b)
        _issue_scatters(b)
        _issue_dst(jnp.minimum(k + np.int32(2), np.int32(_NCHUNK - 1)), b)

    pltpu.sync_copy(dst_hbm.at[pl.ds(_ebase(np.int32(0)), _C)], dst0)
    _issue_loads(np.int32(0), 0)
    _issue_dst(np.int32(1), 1)
    _body(np.int32(0), 0, wait_scat=False)
    _body(np.int32(1), 1, wait_scat=True)

    def _pipe(j, carry):
        for u in range(2):
            _body(j * np.int32(2) + np.int32(u), u, wait_scat=True)
        return carry

    lax.fori_loop(np.int32(1), np.int32((_NCHUNK - 1) // 2), _pipe, np.int32(0))
    _wait_scatters(1)
    _wait_dst(1)
    _wait_loads(0)
    _compute(0)
    _issue_scatters(0)
    _wait_scatters(0)
    plsc.subcore_barrier()

    for j in range(_SLAB // _C):
        r = row0 + np.int32(j * _C)
        pltpu.sync_copy(spmix.at[pl.ds(r, _C), :], efw0)
        pltpu.sync_copy(efw0, mix_out.at[c, pl.ds(r, _C), :])
        pltpu.sync_copy(spden.at[pl.ds(r, _C)], t0)
        pltpu.sync_copy(t0, den_out.at[pl.ds(c * np.int32(_NPAD) + r, _C)])


def _sc_pass(q2, edge_feats, dst32):
    mesh = plsc.VectorSubcoreMesh(core_axis_name="c", subcore_axis_name="s",
                                  num_cores=_NC, num_subcores=_NS)
    fn = pl.kernel(
        _sc_body,
        out_type=[
            jax.ShapeDtypeStruct((_NC, _NPAD, D), jnp.float32),
            jax.ShapeDtypeStruct((_NC * _NPAD,), jnp.float32),
        ],
        mesh=mesh,
        scratch_types=[
            pltpu.VMEM((_C,), jnp.int32),
            pltpu.VMEM((_C,), jnp.int32),
            pltpu.VMEM((_C,), jnp.int32),
            pltpu.VMEM((_C,), jnp.int32),
            pltpu.VMEM((_C, D), jnp.float32),
            pltpu.VMEM((_C, D), jnp.float32),
            pltpu.VMEM((_C, D // 2), jnp.int32),
            pltpu.VMEM((_C, D // 2), jnp.int32),
            pltpu.VMEM((_C,), jnp.float32),
            pltpu.VMEM((_C,), jnp.float32),
            pltpu.VMEM_SHARED((_NPAD, D), jnp.float32),
            pltpu.VMEM_SHARED((_NPAD,), jnp.float32),
        ] + [pltpu.SemaphoreType.DMA] * 10,
        compiler_params=pltpu.CompilerParams(use_tc_tiling_on_sc=False),
    )
    return fn(q2, edge_feats, dst32)


_BLK = 1024


def _q2_body(dstf_ref, wattn2_ref, q2_ref):
    q2_ref[...] = lax.dot_general(
        dstf_ref[...], wattn2_ref[...], (((1,), (1,)), ((), ())),
        preferred_element_type=jnp.float32)


_PERM = np.zeros((D,), np.int32)
for _k in range(D // 32):
    for _j in range(16):
        _PERM[32 * _k + 2 * _j] = 32 * _k + _j
        _PERM[32 * _k + 2 * _j + 1] = 32 * _k + 16 + _j


def _dense_q2(dst_feats, W_attn):
    w2 = W_attn[D:, :][_PERM]
    return pl.pallas_call(
        _q2_body,
        grid=(N_NODES // 1000,),
        in_specs=[
            pl.BlockSpec((1000, D), lambda i: (i, _Z)),
            pl.BlockSpec((D, D), lambda i: (_Z, _Z)),
        ],
        out_specs=pl.BlockSpec((1000, D), lambda i: (i, _Z)),
        out_shape=jax.ShapeDtypeStruct((N_NODES, D), jnp.float32),
    )(dst_feats, w2)


def _dense_tail_body(dstf_ref, mixp_ref, denp_ref, wao_ref, wattn_ref,
                     wneis_ref, wagg_ref, wnode_ref, bnode_ref, out_ref):
    dstf = dstf_ref[...]
    mixp = mixp_ref[...]
    denp = denp_ref[...]
    wao = wao_ref[...]
    wattn = wattn_ref[...]

    den = denp[0] + denp[1]
    pos = den > 0
    invd = jnp.where(pos, 1.0 / jnp.where(pos, den, jnp.ones_like(den)),
                     jnp.zeros_like(den))
    sw = jnp.where(pos, jnp.ones_like(den), jnp.zeros_like(den))
    mix2 = (mixp[0] + mixp[1]) * invd

    def mm(a, b):
        return lax.dot_general(a, b, (((1,), (0,)), ((), ())),
                               preferred_element_type=jnp.float32)

    a1 = wao[:, :128].T
    a2 = wao[:, 128:256].T
    b_fold = mm(wattn.T, wao[:, 256:].T)
    pre = mm(dstf * sw, a1) + mm(mix2, a2) + mm(dstf, b_fold)
    attn_out = jnp.tanh(pre)
    agg = jax.nn.relu(mm(attn_out, wneis_ref[...].T))
    feats_t = mm(dstf, wagg_ref[...].T)
    wnode = wnode_ref[...]
    out = mm(feats_t, wnode[:, :128].T) + mm(agg, wnode[:, 128:].T)
    out_ref[...] = out + bnode_ref[...]


def _dense_tail(dstf_pad, mix_parts, den_parts, W_attn_out, W_attn, W_neis,
                W_agg, W_node, b_node):
    out = pl.pallas_call(
        _dense_tail_body,
        grid=(_NPAD // _BLK,),
        in_specs=[
            pl.BlockSpec((_BLK, D), lambda i: (i, _Z)),
            pl.BlockSpec((_NC, _BLK, D), lambda i: (_Z, i, _Z)),
            pl.BlockSpec((_NC, _BLK, 1), lambda i: (_Z, i, _Z)),
            pl.BlockSpec((2 * D, 4 * D), lambda i: (_Z, _Z)),
            pl.BlockSpec((2 * D, D), lambda i: (_Z, _Z)),
            pl.BlockSpec((D, 2 * D), lambda i: (_Z, _Z)),
            pl.BlockSpec((D, D), lambda i: (_Z, _Z)),
            pl.BlockSpec((D, 2 * D), lambda i: (_Z, _Z)),
            pl.BlockSpec((1, D), lambda i: (_Z, _Z)),
        ],
        out_specs=pl.BlockSpec((_BLK, D), lambda i: (i, _Z)),
        out_shape=jax.ShapeDtypeStruct((_NPAD, D), jnp.float32),
    )(dstf_pad, mix_parts, den_parts.reshape(_NC, _NPAD, 1), W_attn_out,
      W_attn, W_neis, W_agg, W_node, b_node.reshape(1, D))
    return out[:N_NODES]


def kernel(src_feats, dst_feats, edge_feats, edge_index, W_attn, W_agg,
           W_neis, W_attn_out, W_node, b_node, W_edge, b_edge):
    del src_feats, W_edge, b_edge
    with jax.enable_x64(False):
        dst32 = edge_index[1].astype(jnp.int32)

        q2 = _dense_q2(dst_feats, W_attn)
        q2i = lax.bitcast_convert_type(
            q2.astype(jnp.bfloat16).reshape(N_NODES, D // 2, 2), jnp.int32)
        mix_parts, den_parts = _sc_pass(q2i, edge_feats, dst32)

        dstf_pad = jnp.pad(dst_feats, ((0, _NPAD - N_NODES), (0, 0)))
        return _dense_tail(dstf_pad, mix_parts, den_parts, W_attn_out,
                           W_attn, W_neis, W_agg, W_node, b_node)


# speedup vs baseline: 1.0330x; 1.0330x over previous
import functools

import jax
import jax.numpy as jnp
import numpy as np
from jax import lax
from jax.experimental import pallas as pl
from jax.experimental.pallas import tpu as pltpu
from jax.experimental.pallas import tpu_sc as plsc

N_NODES = 10000
N_EDGES = 320000
D = 128

_Z = np.int32(0)

_NC = 2
_NS = 16
_L = 16
_NW = _NC * _NS
_EPW = N_EDGES // _NW
_C = 80
_NCHUNK = _EPW // _C
_NPAD = 10240
_SLAB = _NPAD // _NS


def _sc_body(q2_hbm, ef_hbm, dst_hbm, mix_out, den_out,
             dst0, dst1, dsts0, dsts1, efw0, efw1, q2r0, q2r1, t0, t1,
             spmix, spden,
             semd0, semd1, semg0, semg1, seme0, seme1,
             semt0, semt1, semm0, semm1):
    c = lax.axis_index("c")
    s = lax.axis_index("s")
    wid = s * np.int32(_NC) + c
    row0 = s * np.int32(_SLAB)
    dst_b = (dst0, dst1)
    dsts_b = (dsts0, dsts1)
    efw_b = (efw0, efw1)
    q2r_b = (q2r0, q2r1)
    t_b = (t0, t1)
    semd_b = (semd0, semd1)
    semg_b = (semg0, semg1)
    seme_b = (seme0, seme1)
    semt_b = (semt0, semt1)
    semm_b = (semm0, semm1)

    def _zrow(j, carry):
        for k in range(8):
            efw0[j, pl.ds(k * _L, _L)] = jnp.zeros((_L,), jnp.float32)
        return carry

    lax.fori_loop(np.int32(0), np.int32(_C), _zrow, np.int32(0))
    for g in range(_C // _L):
        t0[pl.ds(g * _L, _L)] = jnp.zeros((_L,), jnp.float32)
    for j in range(_SLAB // _C):
        pltpu.sync_copy(efw0, spmix.at[pl.ds(row0 + np.int32(j * _C), _C), :])
        pltpu.sync_copy(t0, spden.at[pl.ds(row0 + np.int32(j * _C), _C)])
    plsc.subcore_barrier()

    base_w = wid * np.int32(_EPW)
    lane = lax.iota(jnp.int32, _L)
    perms = [(lane + np.int32(sh)) & np.int32(_L - 1) for sh in (8, 4, 2, 1)]
    dnums = lax.GatherDimensionNumbers(
        offset_dims=(), collapsed_slice_dims=(0,), start_index_map=(0,))

    def _hsum(v):
        for pm in perms:
            v = v + lax.gather(v, pm[:, None], dnums, (1,),
                               mode=lax.GatherScatterMode.PROMISE_IN_BOUNDS)
        return v

    def _ebase(k):
        return base_w + k * np.int32(_C)

    def _issue_loads(k, b):
        pltpu.async_copy(q2_hbm.at[dst_b[b]], q2r_b[b], semg_b[b])
        pltpu.async_copy(ef_hbm.at[pl.ds(_ebase(k), _C), :], efw_b[b], seme_b[b])

    def _wait_loads(b):
        pltpu.make_async_copy(q2_hbm.at[dst_b[b]], q2r_b[b], semg_b[b]).wait()
        pltpu.make_async_copy(ef_hbm.at[pl.ds(np.int32(0), _C), :], efw_b[b],
                              seme_b[b]).wait()

    def _issue_dst(k, b):
        pltpu.async_copy(dst_hbm.at[pl.ds(_ebase(k), _C)], dst_b[b], semd_b[b])

    def _wait_dst(b):
        pltpu.make_async_copy(dst_hbm.at[pl.ds(np.int32(0), _C)], dst_b[b],
                              semd_b[b]).wait()

    def _wait_scatters(b):
        pltpu.make_async_copy(t_b[b], spden.at[dsts_b[b]], semt_b[b]).wait()
        pltpu.make_async_copy(efw_b[b], spmix.at[dsts_b[b]], semm_b[b]).wait()

    def _compute(b):
        efw_v = efw_b[b]
        q2r_v = q2r_b[b]
        t_v = t_b[b]

        def _grp(g, cc):
            e0 = g * np.int32(_L)

            def _edge(e, tvec):
                row = e0 + e
                r = [efw_v[row, pl.ds(k * _L, _L)] for k in range(D // _L)]
                acc = r[0] * q2r_v[row, pl.ds(0, _L)]
                for k in range(1, D // _L):
                    acc = acc + r[k] * q2r_v[row, pl.ds(k * _L, _L)]
                exv = jnp.exp(jnp.minimum(_hsum(acc), np.float32(80.0)))
                for k in range(D // _L):
                    efw_v[row, pl.ds(k * _L, _L)] = r[k] * exv
                return jnp.where(lane == e, exv, tvec)

            tvec = lax.fori_loop(np.int32(0), np.int32(_L), _edge,
                                 jnp.zeros((_L,), jnp.float32))
            t_v[pl.ds(e0, _L)] = tvec
            return cc

        lax.fori_loop(np.int32(0), np.int32(_C // _L), _grp, np.int32(0))

    def _issue_scatters(b):
        for g in range(_C // _L):
            sl = pl.ds(g * _L, _L)
            dsts_b[b][sl] = dst_b[b][sl]
        pltpu.async_copy(t_b[b], spden.at[dsts_b[b]], semt_b[b], add=True)
        pltpu.async_copy(efw_b[b], spmix.at[dsts_b[b]], semm_b[b], add=True)

    def _body(k, b, wait_scat):
        nb = 1 - b
        if wait_scat:
            _wait_scatters(nb)
        _wait_loads(b)
        _wait_dst(nb)
        _issue_loads(k + np.int32(1), nb)
        _compute(b)
        _issue_scatters(b)
        _issue_dst(jnp.minimum(k + np.int32(2), np.int32(_NCHUNK - 1)), b)

    pltpu.sync_copy(dst_hbm.at[pl.ds(_ebase(np.int32(0)), _C)], dst0)
    _issue_loads(np.int32(0), 0)
    _issue_dst(np.int32(1), 1)
    _body(np.int32(0), 0, wait_scat=False)
    _body(np.int32(1), 1, wait_scat=True)

    def _pipe(j, carry):
        for u in range(2):
            _body(j * np.int32(2) + np.int32(u), u, wait_scat=True)
        return carry

    lax.fori_loop(np.int32(1), np.int32((_NCHUNK - 1) // 2), _pipe, np.int32(0))
    _wait_scatters(1)
    _wait_dst(1)
    _wait_loads(0)
    _compute(0)
    _issue_scatters(0)
    _wait_scatters(0)
    plsc.subcore_barrier()

    for j in range(_SLAB // _C):
        r = row0 + np.int32(j * _C)
        pltpu.sync_copy(spmix.at[pl.ds(r, _C), :], efw0)
        pltpu.sync_copy(efw0, mix_out.at[c, pl.ds(r, _C), :])
        pltpu.sync_copy(spden.at[pl.ds(r, _C)], t0)
        pltpu.sync_copy(t0, den_out.at[pl.ds(c * np.int32(_NPAD) + r, _C)])


def _sc_pass(q2, edge_feats, dst32):
    mesh = plsc.VectorSubcoreMesh(core_axis_name="c", subcore_axis_name="s",
                                  num_cores=_NC, num_subcores=_NS)
    fn = pl.kernel(
        _sc_body,
        out_type=[
            jax.ShapeDtypeStruct((_NC, _NPAD, D), jnp.float32),
            jax.ShapeDtypeStruct((_NC * _NPAD,), jnp.float32),
        ],
        mesh=mesh,
        scratch_types=[
            pltpu.VMEM((_C,), jnp.int32),
            pltpu.VMEM((_C,), jnp.int32),
            pltpu.VMEM((_C,), jnp.int32),
            pltpu.VMEM((_C,), jnp.int32),
            pltpu.VMEM((_C, D), jnp.float32),
            pltpu.VMEM((_C, D), jnp.float32),
            pltpu.VMEM((_C, D), jnp.float32),
            pltpu.VMEM((_C, D), jnp.float32),
            pltpu.VMEM((_C,), jnp.float32),
            pltpu.VMEM((_C,), jnp.float32),
            pltpu.VMEM_SHARED((_NPAD, D), jnp.float32),
            pltpu.VMEM_SHARED((_NPAD,), jnp.float32),
        ] + [pltpu.SemaphoreType.DMA] * 10,
        compiler_params=pltpu.CompilerParams(use_tc_tiling_on_sc=False),
    )
    return fn(q2, edge_feats, dst32)


_BLK = 1024


def _q2_body(dstf_ref, wattn2_ref, q2_ref):
    q2_ref[...] = lax.dot_general(
        dstf_ref[...], wattn2_ref[...], (((1,), (1,)), ((), ())),
        preferred_element_type=jnp.float32)


def _dense_q2(dst_feats, W_attn):
    w2 = W_attn[D:, :]
    return pl.pallas_call(
        _q2_body,
        grid=(N_NODES // 1000,),
        in_specs=[
            pl.BlockSpec((1000, D), lambda i: (i, _Z)),
            pl.BlockSpec((D, D), lambda i: (_Z, _Z)),
        ],
        out_specs=pl.BlockSpec((1000, D), lambda i: (i, _Z)),
        out_shape=jax.ShapeDtypeStruct((N_NODES, D), jnp.float32),
    )(dst_feats, w2)


def _dense_tail_body(dstf_ref, mixp_ref, denp_ref, wao_ref, wattn_ref,
                     wneis_ref, wagg_ref, wnode_ref, bnode_ref, out_ref):
    dstf = dstf_ref[...]
    mixp = mixp_ref[...]
    denp = denp_ref[...]
    wao = wao_ref[...]
    wattn = wattn_ref[...]

    den = denp[0] + denp[1]
    pos = den > 0
    invd = jnp.where(pos, 1.0 / jnp.where(pos, den, jnp.ones_like(den)),
                     jnp.zeros_like(den))
    sw = jnp.where(pos, jnp.ones_like(den), jnp.zeros_like(den))
    mix2 = (mixp[0] + mixp[1]) * invd

    def mm(a, b):
        return lax.dot_general(a, b, (((1,), (0,)), ((), ())),
                               preferred_element_type=jnp.float32)

    a1 = wao[:, :128].T
    a2 = wao[:, 128:256].T
    b_fold = mm(wattn.T, wao[:, 256:].T)
    pre = mm(dstf * sw, a1) + mm(mix2, a2) + mm(dstf, b_fold)
    attn_out = jnp.tanh(pre)
    agg = jax.nn.relu(mm(attn_out, wneis_ref[...].T))
    feats_t = mm(dstf, wagg_ref[...].T)
    wnode = wnode_ref[...]
    out = mm(feats_t, wnode[:, :128].T) + mm(agg, wnode[:, 128:].T)
    out_ref[...] = out + bnode_ref[...]


def _dense_tail(dstf_pad, mix_parts, den_parts, W_attn_out, W_attn, W_neis,
                W_agg, W_node, b_node):
    out = pl.pallas_call(
        _dense_tail_body,
        grid=(_NPAD // _BLK,),
        in_specs=[
            pl.BlockSpec((_BLK, D), lambda i: (i, _Z)),
            pl.BlockSpec((_NC, _BLK, D), lambda i: (_Z, i, _Z)),
            pl.BlockSpec((_NC, _BLK, 1), lambda i: (_Z, i, _Z)),
            pl.BlockSpec((2 * D, 4 * D), lambda i: (_Z, _Z)),
            pl.BlockSpec((2 * D, D), lambda i: (_Z, _Z)),
            pl.BlockSpec((D, 2 * D), lambda i: (_Z, _Z)),
            pl.BlockSpec((D, D), lambda i: (_Z, _Z)),
            pl.BlockSpec((D, 2 * D), lambda i: (_Z, _Z)),
            pl.BlockSpec((1, D), lambda i: (_Z, _Z)),
        ],
        out_specs=pl.BlockSpec((_BLK, D), lambda i: (i, _Z)),
        out_shape=jax.ShapeDtypeStruct((_NPAD, D), jnp.float32),
    )(dstf_pad, mix_parts, den_parts.reshape(_NC, _NPAD, 1), W_attn_out,
      W_attn, W_neis, W_agg, W_node, b_node.reshape(1, D))
    return out[:N_NODES]


def kernel(src_feats, dst_feats, edge_feats, edge_index, W_attn, W_agg,
           W_neis, W_attn_out, W_node, b_node, W_edge, b_edge):
    del src_feats, W_edge, b_edge
    with jax.enable_x64(False):
        dst32 = edge_index[1].astype(jnp.int32)

        q2 = _dense_q2(dst_feats, W_attn)
        mix_parts, den_parts = _sc_pass(q2, edge_feats, dst32)

        dstf_pad = jnp.pad(dst_feats, ((0, _NPAD - N_NODES), (0, 0)))
        return _dense_tail(dstf_pad, mix_parts, den_parts, W_attn_out,
                           W_attn, W_neis, W_agg, W_node, b_node)


# speedup vs baseline: 1.0423x; 1.0091x over previous
import functools

import jax
import jax.numpy as jnp
import numpy as np
from jax import lax
from jax.experimental import pallas as pl
from jax.experimental.pallas import tpu as pltpu
from jax.experimental.pallas import tpu_sc as plsc

N_NODES = 10000
N_EDGES = 320000
D = 128

_Z = np.int32(0)

_NC = 2
_NS = 16
_L = 16
_NW = _NC * _NS
_EPW = N_EDGES // _NW
_C = 80
_NCHUNK = _EPW // _C
_NPAD = 10240
_SLAB = _NPAD // _NS


def _sc_body(q2_hbm, ef_hbm, dst_hbm, mix_out, den_out,
             dst0, dst1, dsts0, dsts1, efw0, efw1, q2r0, q2r1, t0, t1,
             spmix, spden,
             semd0, semd1, semg0, semg1, seme0, seme1,
             semt0, semt1, semm0, semm1):
    c = lax.axis_index("c")
    s = lax.axis_index("s")
    wid = s * np.int32(_NC) + c
    row0 = s * np.int32(_SLAB)
    dst_b = (dst0, dst1)
    dsts_b = (dsts0, dsts1)
    efw_b = (efw0, efw1)
    q2r_b = (q2r0, q2r1)
    t_b = (t0, t1)
    semd_b = (semd0, semd1)
    semg_b = (semg0, semg1)
    seme_b = (seme0, seme1)
    semt_b = (semt0, semt1)
    semm_b = (semm0, semm1)

    def _zrow(j, carry):
        for k in range(8):
            efw0[j, pl.ds(k * _L, _L)] = jnp.zeros((_L,), jnp.float32)
        return carry

    lax.fori_loop(np.int32(0), np.int32(_C), _zrow, np.int32(0))
    for g in range(_C // _L):
        t0[pl.ds(g * _L, _L)] = jnp.zeros((_L,), jnp.float32)
    for j in range(_SLAB // _C):
        pltpu.sync_copy(efw0, spmix.at[pl.ds(row0 + np.int32(j * _C), _C), :])
        pltpu.sync_copy(t0, spden.at[pl.ds(row0 + np.int32(j * _C), _C)])
    plsc.subcore_barrier()

    base_w = wid * np.int32(_EPW)
    lane = lax.iota(jnp.int32, _L)
    perms = [(lane + np.int32(sh)) & np.int32(_L - 1) for sh in (8, 4, 2, 1)]
    dnums = lax.GatherDimensionNumbers(
        offset_dims=(), collapsed_slice_dims=(0,), start_index_map=(0,))

    def _hsum(v):
        for pm in perms:
            v = v + lax.gather(v, pm[:, None], dnums, (1,),
                               mode=lax.GatherScatterMode.PROMISE_IN_BOUNDS)
        return v

    def _ebase(k):
        return base_w + k * np.int32(_C)

    def _issue_loads(k, b):
        pltpu.async_copy(q2_hbm.at[dst_b[b]], q2r_b[b], semg_b[b])
        pltpu.async_copy(ef_hbm.at[pl.ds(_ebase(k), _C), :], efw_b[b], seme_b[b])

    def _wait_loads(b):
        pltpu.make_async_copy(q2_hbm.at[dst_b[b]], q2r_b[b], semg_b[b]).wait()
        pltpu.make_async_copy(ef_hbm.at[pl.ds(np.int32(0), _C), :], efw_b[b],
                              seme_b[b]).wait()

    def _issue_dst(k, b):
        pltpu.async_copy(dst_hbm.at[pl.ds(_ebase(k), _C)], dst_b[b], semd_b[b])

    def _wait_dst(b):
        pltpu.make_async_copy(dst_hbm.at[pl.ds(np.int32(0), _C)], dst_b[b],
                              semd_b[b]).wait()

    def _wait_scatters(b):
        pltpu.make_async_copy(t_b[b], spden.at[dsts_b[b]], semt_b[b]).wait()
        pltpu.make_async_copy(efw_b[b], spmix.at[dsts_b[b]], semm_b[b]).wait()

    def _compute(b):
        efw_v = efw_b[b]
        q2r_v = q2r_b[b]
        t_v = t_b[b]

        def _grp(g, cc):
            e0 = g * np.int32(_L)

            def _edge(e, tvec):
                row = e0 + e
                r = [efw_v[row, pl.ds(k * _L, _L)] for k in range(D // _L)]
                acc = r[0] * q2r_v[row, pl.ds(0, _L)]
                for k in range(1, D // _L):
                    acc = acc + r[k] * q2r_v[row, pl.ds(k * _L, _L)]
                exv = jnp.exp(jnp.minimum(_hsum(acc), np.float32(80.0)))
                for k in range(D // _L):
                    efw_v[row, pl.ds(k * _L, _L)] = r[k] * exv
                return jnp.where(lane == e, exv, tvec)

            tvec = lax.fori_loop(np.int32(0), np.int32(_L), _edge,
                                 jnp.zeros((_L,), jnp.float32))
            t_v[pl.ds(e0, _L)] = tvec
            return cc

        lax.fori_loop(np.int32(0), np.int32(_C // _L), _grp, np.int32(0))

    def _issue_scatters(b):
        for g in range(_C // _L):
            sl = pl.ds(g * _L, _L)
            dsts_b[b][sl] = dst_b[b][sl]
        pltpu.async_copy(t_b[b], spden.at[dsts_b[b]], semt_b[b], add=True)
        pltpu.async_copy(efw_b[b], spmix.at[dsts_b[b]], semm_b[b], add=True)

    def _body(k, b, wait_scat):
        nb = 1 - b
        if wait_scat:
            _wait_scatters(nb)
        _wait_loads(b)
        _wait_dst(nb)
        _issue_loads(k + np.int32(1), nb)
        _compute(b)
        _issue_scatters(b)
        _issue_dst(jnp.minimum(k + np.int32(2), np.int32(_NCHUNK - 1)), b)

    pltpu.sync_copy(dst_hbm.at[pl.ds(_ebase(np.int32(0)), _C)], dst0)
    _issue_loads(np.int32(0), 0)
    _issue_dst(np.int32(1), 1)
    _body(np.int32(0), 0, wait_scat=False)
    _body(np.int32(1), 1, wait_scat=True)

    def _pipe(j, carry):
        for u in range(2):
            _body(j * np.int32(2) + np.int32(u), u, wait_scat=True)
        return carry

    lax.fori_loop(np.int32(1), np.int32((_NCHUNK - 1) // 2), _pipe, np.int32(0))
    _wait_scatters(1)
    _wait_dst(1)
    _wait_loads(0)
    _compute(0)
    _issue_scatters(0)
    _wait_scatters(0)
    plsc.subcore_barrier()

    for j in range(_SLAB // _C):
        r = row0 + np.int32(j * _C)
        pltpu.sync_copy(spmix.at[pl.ds(r, _C), :], efw0)
        pltpu.sync_copy(efw0, mix_out.at[c, pl.ds(r, _C), :])
        pltpu.sync_copy(spden.at[pl.ds(r, _C)], t0)
        pltpu.sync_copy(t0, den_out.at[pl.ds(c * np.int32(_NPAD) + r, _C)])


def _sc_pass(q2, edge_feats, dst32):
    mesh = plsc.VectorSubcoreMesh(core_axis_name="c", subcore_axis_name="s",
                                  num_cores=_NC, num_subcores=_NS)
    fn = pl.kernel(
        _sc_body,
        out_type=[
            jax.ShapeDtypeStruct((_NC, _NPAD, D), jnp.float32),
            jax.ShapeDtypeStruct((_NC * _NPAD,), jnp.float32),
        ],
        mesh=mesh,
        scratch_types=[
            pltpu.VMEM((_C,), jnp.int32),
            pltpu.VMEM((_C,), jnp.int32),
            pltpu.VMEM((_C,), jnp.int32),
            pltpu.VMEM((_C,), jnp.int32),
            pltpu.VMEM((_C, D), jnp.float32),
            pltpu.VMEM((_C, D), jnp.float32),
            pltpu.VMEM((_C, D), jnp.float32),
            pltpu.VMEM((_C, D), jnp.float32),
            pltpu.VMEM((_C,), jnp.float32),
            pltpu.VMEM((_C,), jnp.float32),
            pltpu.VMEM_SHARED((_NPAD, D), jnp.float32),
            pltpu.VMEM_SHARED((_NPAD,), jnp.float32),
        ] + [pltpu.SemaphoreType.DMA] * 10,
        compiler_params=pltpu.CompilerParams(use_tc_tiling_on_sc=False),
    )
    return fn(q2, edge_feats, dst32)


_BLK = 1000


def _q2_body(dstf_ref, wattn2_ref, q2_ref):
    q2_ref[...] = lax.dot_general(
        dstf_ref[...], wattn2_ref[...], (((1,), (1,)), ((), ())),
        preferred_element_type=jnp.float32)


def _dense_q2(dst_feats, W_attn):
    w2 = W_attn[D:, :]
    return pl.pallas_call(
        _q2_body,
        grid=(N_NODES // 1000,),
        in_specs=[
            pl.BlockSpec((1000, D), lambda i: (i, _Z)),
            pl.BlockSpec((D, D), lambda i: (_Z, _Z)),
        ],
        out_specs=pl.BlockSpec((1000, D), lambda i: (i, _Z)),
        out_shape=jax.ShapeDtypeStruct((N_NODES, D), jnp.float32),
    )(dst_feats, w2)


def _dense_tail_body(dstf_ref, mixp_ref, denp_ref, wao_ref, wattn_ref,
                     wneis_ref, wagg_ref, wnode_ref, bnode_ref, out_ref):
    dstf = dstf_ref[...]
    mixp = mixp_ref[...]
    denp = denp_ref[...]
    wao = wao_ref[...]
    wattn = wattn_ref[...]

    den = denp[0] + denp[1]
    pos = den > 0
    invd = jnp.where(pos, 1.0 / jnp.where(pos, den, jnp.ones_like(den)),
                     jnp.zeros_like(den))
    sw = jnp.where(pos, jnp.ones_like(den), jnp.zeros_like(den))
    mix2 = (mixp[0] + mixp[1]) * invd

    def mm(a, b):
        return lax.dot_general(a, b, (((1,), (0,)), ((), ())),
                               preferred_element_type=jnp.float32)

    a1 = wao[:, :128].T
    a2 = wao[:, 128:256].T
    b_fold = mm(wattn.T, wao[:, 256:].T)
    pre = mm(dstf * sw, a1) + mm(mix2, a2) + mm(dstf, b_fold)
    attn_out = jnp.tanh(pre)
    agg = jax.nn.relu(mm(attn_out, wneis_ref[...].T))
    feats_t = mm(dstf, wagg_ref[...].T)
    wnode = wnode_ref[...]
    out = mm(feats_t, wnode[:, :128].T) + mm(agg, wnode[:, 128:].T)
    out_ref[...] = out + bnode_ref[...]


def _dense_tail(dst_feats, mix_parts, den_parts, W_attn_out, W_attn, W_neis,
                W_agg, W_node, b_node):
    out = pl.pallas_call(
        _dense_tail_body,
        grid=(N_NODES // _BLK,),
        in_specs=[
            pl.BlockSpec((_BLK, D), lambda i: (i, _Z)),
            pl.BlockSpec((_NC, _BLK, D), lambda i: (_Z, i, _Z)),
            pl.BlockSpec((_NC, _BLK, 1), lambda i: (_Z, i, _Z)),
            pl.BlockSpec((2 * D, 4 * D), lambda i: (_Z, _Z)),
            pl.BlockSpec((2 * D, D), lambda i: (_Z, _Z)),
            pl.BlockSpec((D, 2 * D), lambda i: (_Z, _Z)),
            pl.BlockSpec((D, D), lambda i: (_Z, _Z)),
            pl.BlockSpec((D, 2 * D), lambda i: (_Z, _Z)),
            pl.BlockSpec((1, D), lambda i: (_Z, _Z)),
        ],
        out_specs=pl.BlockSpec((_BLK, D), lambda i: (i, _Z)),
        out_shape=jax.ShapeDtypeStruct((N_NODES, D), jnp.float32),
    )(dst_feats, mix_parts, den_parts.reshape(_NC, _NPAD, 1), W_attn_out,
      W_attn, W_neis, W_agg, W_node, b_node.reshape(1, D))
    return out


def kernel(src_feats, dst_feats, edge_feats, edge_index, W_attn, W_agg,
           W_neis, W_attn_out, W_node, b_node, W_edge, b_edge):
    del src_feats, W_edge, b_edge
    with jax.enable_x64(False):
        dst32 = edge_index[1].astype(jnp.int32)

        q2 = _dense_q2(dst_feats, W_attn)
        mix_parts, den_parts = _sc_pass(q2, edge_feats, dst32)

        return _dense_tail(dst_feats, mix_parts, den_parts, W_attn_out,
                           W_attn, W_neis, W_agg, W_node, b_node)


# speedup vs baseline: 1.0427x; 1.0004x over previous
import jax
import jax.numpy as jnp
import numpy as np
from jax import lax
from jax.experimental import pallas as pl
from jax.experimental.pallas import tpu as pltpu
from jax.experimental.pallas import tpu_sc as plsc

N_NODES = 10000
N_EDGES = 320000
D = 128

_Z = np.int32(0)

_NC = 2
_NS = 16
_L = 16
_NW = _NC * _NS
_EPW = N_EDGES // _NW
_C = 80
_NCHUNK = _EPW // _C
_NPAD = 10240
_SLAB = _NPAD // _NS


def _sc_body(q2_hbm, ef_hbm, dst_hbm, mix_out, den_out,
             dst0, dst1, dsts0, dsts1, efw0, efw1, q2r0, q2r1, t0, t1,
             spmix, spden,
             semd0, semd1, semg0, semg1, seme0, seme1,
             semt0, semt1, semm0, semm1):
    c = lax.axis_index("c")
    s = lax.axis_index("s")
    wid = s * np.int32(_NC) + c
    row0 = s * np.int32(_SLAB)
    dst_b = (dst0, dst1)
    dsts_b = (dsts0, dsts1)
    efw_b = (efw0, efw1)
    q2r_b = (q2r0, q2r1)
    t_b = (t0, t1)
    semd_b = (semd0, semd1)
    semg_b = (semg0, semg1)
    seme_b = (seme0, seme1)
    semt_b = (semt0, semt1)
    semm_b = (semm0, semm1)

    def _zrow(j, carry):
        for k in range(8):
            efw0[j, pl.ds(k * _L, _L)] = jnp.zeros((_L,), jnp.float32)
        return carry

    lax.fori_loop(np.int32(0), np.int32(_C), _zrow, np.int32(0))
    for g in range(_C // _L):
        t0[pl.ds(g * _L, _L)] = jnp.zeros((_L,), jnp.float32)
    for j in range(_SLAB // _C):
        pltpu.sync_copy(efw0, spmix.at[pl.ds(row0 + np.int32(j * _C), _C), :])
        pltpu.sync_copy(t0, spden.at[pl.ds(row0 + np.int32(j * _C), _C)])
    plsc.subcore_barrier()

    base_w = wid * np.int32(_EPW)
    lane = lax.iota(jnp.int32, _L)
    perms = [(lane + np.int32(sh)) & np.int32(_L - 1) for sh in (8, 4, 2, 1)]
    dnums = lax.GatherDimensionNumbers(
        offset_dims=(), collapsed_slice_dims=(0,), start_index_map=(0,))

    def _hsum(v):
        for pm in perms:
            v = v + lax.gather(v, pm[:, None], dnums, (1,),
                               mode=lax.GatherScatterMode.PROMISE_IN_BOUNDS)
        return v

    def _ebase(k):
        return base_w + k * np.int32(_C)

    def _issue_loads(k, b):
        pltpu.async_copy(q2_hbm.at[dst_b[b]], q2r_b[b], semg_b[b])
        pltpu.async_copy(ef_hbm.at[pl.ds(_ebase(k), _C), :], efw_b[b], seme_b[b])

    def _wait_loads(b):
        pltpu.make_async_copy(q2_hbm.at[dst_b[b]], q2r_b[b], semg_b[b]).wait()
        pltpu.make_async_copy(ef_hbm.at[pl.ds(np.int32(0), _C), :], efw_b[b],
                              seme_b[b]).wait()

    def _issue_dst(k, b):
        pltpu.async_copy(dst_hbm.at[pl.ds(_ebase(k), _C)], dst_b[b], semd_b[b])

    def _wait_dst(b):
        pltpu.make_async_copy(dst_hbm.at[pl.ds(np.int32(0), _C)], dst_b[b],
                              semd_b[b]).wait()

    def _wait_scatters(b):
        pltpu.make_async_copy(t_b[b], spden.at[dsts_b[b]], semt_b[b]).wait()
        pltpu.make_async_copy(efw_b[b], spmix.at[dsts_b[b]], semm_b[b]).wait()

    def _compute(b):
        efw_v = efw_b[b]
        q2r_v = q2r_b[b]
        t_v = t_b[b]

        def _grp(g, cc):
            e0 = g * np.int32(_L)

            def _edge(e, tvec):
                row = e0 + e
                r = [efw_v[row, pl.ds(k * _L, _L)] for k in range(D // _L)]
                acc = r[0] * q2r_v[row, pl.ds(0, _L)]
                for k in range(1, D // _L):
                    acc = acc + r[k] * q2r_v[row, pl.ds(k * _L, _L)]
                exv = jnp.exp(jnp.minimum(_hsum(acc), np.float32(80.0)))
                for k in range(D // _L):
                    efw_v[row, pl.ds(k * _L, _L)] = r[k] * exv
                return jnp.where(lane == e, exv, tvec)

            tvec = lax.fori_loop(np.int32(0), np.int32(_L), _edge,
                                 jnp.zeros((_L,), jnp.float32))
            t_v[pl.ds(e0, _L)] = tvec
            return cc

        lax.fori_loop(np.int32(0), np.int32(_C // _L), _grp, np.int32(0))

    def _issue_scatters(b):
        for g in range(_C // _L):
            sl = pl.ds(g * _L, _L)
            dsts_b[b][sl] = dst_b[b][sl]
        pltpu.async_copy(t_b[b], spden.at[dsts_b[b]], semt_b[b], add=True)
        pltpu.async_copy(efw_b[b], spmix.at[dsts_b[b]], semm_b[b], add=True)

    def _body(k, b, wait_scat):
        nb = 1 - b
        if wait_scat:
            _wait_scatters(nb)
        _wait_loads(b)
        _wait_dst(nb)
        _issue_loads(k + np.int32(1), nb)
        _compute(b)
        _issue_scatters(b)
        _issue_dst(jnp.minimum(k + np.int32(2), np.int32(_NCHUNK - 1)), b)

    pltpu.sync_copy(dst_hbm.at[pl.ds(_ebase(np.int32(0)), _C)], dst0)
    _issue_loads(np.int32(0), 0)
    _issue_dst(np.int32(1), 1)
    _body(np.int32(0), 0, wait_scat=False)
    _body(np.int32(1), 1, wait_scat=True)

    def _pipe(j, carry):
        for u in range(2):
            _body(j * np.int32(2) + np.int32(u), u, wait_scat=True)
        return carry

    lax.fori_loop(np.int32(1), np.int32((_NCHUNK - 1) // 2), _pipe, np.int32(0))
    _wait_scatters(1)
    _wait_dst(1)
    _wait_loads(0)
    _compute(0)
    _issue_scatters(0)
    _wait_scatters(0)
    plsc.subcore_barrier()

    for j in range(_SLAB // _C):
        r = row0 + np.int32(j * _C)
        pltpu.sync_copy(spmix.at[pl.ds(r, _C), :], efw0)
        pltpu.sync_copy(efw0, mix_out.at[c, pl.ds(r, _C), :])
        pltpu.sync_copy(spden.at[pl.ds(r, _C)], t0)
        pltpu.sync_copy(t0, den_out.at[pl.ds(c * np.int32(_NPAD) + r, _C)])


def _sc_pass(q2, edge_feats, dst32):
    mesh = plsc.VectorSubcoreMesh(core_axis_name="c", subcore_axis_name="s",
                                  num_cores=_NC, num_subcores=_NS)
    fn = pl.kernel(
        _sc_body,
        out_type=[
            jax.ShapeDtypeStruct((_NC, _NPAD, D), jnp.float32),
            jax.ShapeDtypeStruct((_NC * _NPAD,), jnp.float32),
        ],
        mesh=mesh,
        scratch_types=[
            pltpu.VMEM((_C,), jnp.int32),
            pltpu.VMEM((_C,), jnp.int32),
            pltpu.VMEM((_C,), jnp.int32),
            pltpu.VMEM((_C,), jnp.int32),
            pltpu.VMEM((_C, D), jnp.float32),
            pltpu.VMEM((_C, D), jnp.float32),
            pltpu.VMEM((_C, D), jnp.float32),
            pltpu.VMEM((_C, D), jnp.float32),
            pltpu.VMEM((_C,), jnp.float32),
            pltpu.VMEM((_C,), jnp.float32),
            pltpu.VMEM_SHARED((_NPAD, D), jnp.float32),
            pltpu.VMEM_SHARED((_NPAD,), jnp.float32),
        ] + [pltpu.SemaphoreType.DMA] * 10,
        compiler_params=pltpu.CompilerParams(use_tc_tiling_on_sc=False),
    )
    return fn(q2, edge_feats, dst32)


_BLK = 1000


def _q2_body(dstf_ref, wattn2_ref, q2_ref):
    q2_ref[...] = lax.dot_general(
        dstf_ref[...], wattn2_ref[...], (((1,), (1,)), ((), ())),
        preferred_element_type=jnp.float32)


def _dense_q2(dst_feats, W_attn):
    w2 = W_attn[D:, :]
    return pl.pallas_call(
        _q2_body,
        grid=(N_NODES // 1000,),
        in_specs=[
            pl.BlockSpec((1000, D), lambda i: (i, _Z)),
            pl.BlockSpec((D, D), lambda i: (_Z, _Z)),
        ],
        out_specs=pl.BlockSpec((1000, D), lambda i: (i, _Z)),
        out_shape=jax.ShapeDtypeStruct((N_NODES, D), jnp.float32),
    )(dst_feats, w2)


def _dense_tail_body(dstf_ref, mixp_ref, denp_ref, wao_ref, wattn_ref,
                     wneis_ref, wagg_ref, wnode_ref, bnode_ref, out_ref):
    dstf = dstf_ref[...]
    mixp = mixp_ref[...]
    denp = denp_ref[...]
    wao = wao_ref[...]
    wattn = wattn_ref[...]

    den = denp[0] + denp[1]
    pos = den > 0
    invd = jnp.where(pos, 1.0 / jnp.where(pos, den, jnp.ones_like(den)),
                     jnp.zeros_like(den))
    sw = jnp.where(pos, jnp.ones_like(den), jnp.zeros_like(den))
    mix2 = (mixp[0] + mixp[1]) * invd

    def mm(a, b):
        return lax.dot_general(a, b, (((1,), (0,)), ((), ())),
                               preferred_element_type=jnp.float32)

    a1 = wao[:, :128].T
    a2 = wao[:, 128:256].T
    b_fold = mm(wattn.T, wao[:, 256:].T)
    pre = mm(dstf * sw, a1) + mm(mix2, a2) + mm(dstf, b_fold)
    attn_out = jnp.tanh(pre)
    agg = jax.nn.relu(mm(attn_out, wneis_ref[...].T))
    feats_t = mm(dstf, wagg_ref[...].T)
    wnode = wnode_ref[...]
    out = mm(feats_t, wnode[:, :128].T) + mm(agg, wnode[:, 128:].T)
    out_ref[...] = out + bnode_ref[...]


def _dense_tail(dst_feats, mix_parts, den_parts, W_attn_out, W_attn, W_neis,
                W_agg, W_node, b_node):
    out = pl.pallas_call(
        _dense_tail_body,
        grid=(N_NODES // _BLK,),
        in_specs=[
            pl.BlockSpec((_BLK, D), lambda i: (i, _Z)),
            pl.BlockSpec((_NC, _BLK, D), lambda i: (_Z, i, _Z)),
            pl.BlockSpec((_NC, _BLK, 1), lambda i: (_Z, i, _Z)),
            pl.BlockSpec((2 * D, 4 * D), lambda i: (_Z, _Z)),
            pl.BlockSpec((2 * D, D), lambda i: (_Z, _Z)),
            pl.BlockSpec((D, 2 * D), lambda i: (_Z, _Z)),
            pl.BlockSpec((D, D), lambda i: (_Z, _Z)),
            pl.BlockSpec((D, 2 * D), lambda i: (_Z, _Z)),
            pl.BlockSpec((1, D), lambda i: (_Z, _Z)),
        ],
        out_specs=pl.BlockSpec((_BLK, D), lambda i: (i, _Z)),
        out_shape=jax.ShapeDtypeStruct((N_NODES, D), jnp.float32),
    )(dst_feats, mix_parts, den_parts.reshape(_NC, _NPAD, 1), W_attn_out,
      W_attn, W_neis, W_agg, W_node, b_node.reshape(1, D))
    return out


def kernel(src_feats, dst_feats, edge_feats, edge_index, W_attn, W_agg,
           W_neis, W_attn_out, W_node, b_node, W_edge, b_edge):
    del src_feats, W_edge, b_edge
    with jax.enable_x64(False):
        dst32 = edge_index[1].astype(jnp.int32)

        q2 = _dense_q2(dst_feats, W_attn)
        mix_parts, den_parts = _sc_pass(q2, edge_feats, dst32)

        return _dense_tail(dst_feats, mix_parts, den_parts, W_attn_out,
                           W_attn, W_neis, W_agg, W_node, b_node)
